# Initial kernel scaffold; baseline (speedup 1.0000x reference)
import functools

import jax
import jax.numpy as jnp
from jax import lax
from jax.experimental import pallas as pl
from jax.experimental.pallas import tpu as pltpu
from jax.experimental.pallas import tpu_sc as plsc

N = 160000
D = 256
NUM_SEGMENTS = 10000

NUM_CORES = 2
NUM_SUBCORES = 16
DH = D // NUM_CORES
RPT = N // NUM_SUBCORES
RB = 400
NIT = RPT // RB
SCB = 100
NSC = RB // SCB
SEGB = NUM_SEGMENTS // NUM_SUBCORES
ZR = 25
WB = 125


def _seg_sum_body(x_hbm, segs_hbm, out_hbm, acc_sh, xbuf, idxbuf, zbuf):
    c = lax.axis_index("c")
    s = lax.axis_index("s")
    col0 = c * DH
    zero16 = jnp.zeros((16,), jnp.float32)

    def zero_row(r, carry):
        def zero_lane(j, carry2):
            zbuf[r, pl.ds(j * 16, 16)] = zero16
            return carry2
        return lax.fori_loop(0, DH // 16, zero_lane, carry)

    lax.fori_loop(0, ZR, zero_row, 0)

    def zero_copy(z, carry):
        pltpu.sync_copy(zbuf, acc_sh.at[pl.ds(s * SEGB + z * ZR, ZR)])
        return carry

    lax.fori_loop(0, SEGB // ZR, zero_copy, 0)
    plsc.subcore_barrier()

    def body(it, carry):
        r0 = s * RPT + it * RB
        pltpu.sync_copy(x_hbm.at[pl.ds(r0, RB), pl.ds(col0, DH)], xbuf)
        pltpu.sync_copy(segs_hbm.at[pl.ds(s * (RPT // SCB) + it * NSC, NSC)],
                        idxbuf)
        for b in range(NSC):
            pltpu.sync_copy(xbuf.at[pl.ds(b * SCB, SCB)],
                            acc_sh.at[idxbuf.at[b]], add=True)
        return carry

    lax.fori_loop(0, NIT, body, 0)
    plsc.subcore_barrier()

    def wb(w, carry):
        g0 = s * SEGB + w * WB
        pltpu.sync_copy(acc_sh.at[pl.ds(g0, WB)], xbuf.at[pl.ds(0, WB)])
        pltpu.sync_copy(xbuf.at[pl.ds(0, WB)],
                        out_hbm.at[pl.ds(g0, WB), pl.ds(col0, DH)])
        return carry

    lax.fori_loop(0, SEGB // WB, wb, 0)


@jax.jit
def _seg_sum(xs, segs_r):
    f = pl.kernel(
        _seg_sum_body,
        out_type=jax.ShapeDtypeStruct((NUM_SEGMENTS, D), jnp.float32),
        mesh=plsc.VectorSubcoreMesh(core_axis_name="c", subcore_axis_name="s"),
        scratch_types=[
            pltpu.VMEM_SHARED((NUM_SEGMENTS, DH), jnp.float32),
            pltpu.VMEM((RB, DH), jnp.float32),
            pltpu.VMEM((NSC, SCB), jnp.int32),
            pltpu.VMEM((ZR, DH), jnp.float32),
        ],
    )
    return f(xs, segs_r)


def kernel(x, segs):
    xs = jnp.squeeze(x, axis=0)
    segs_r = jnp.reshape(segs, (N // SCB, SCB))
    y = _seg_sum(xs, segs_r)
    return jnp.expand_dims(y, axis=0)


# speedup vs baseline: 4.0817x; 4.0817x over previous
import jax
import jax.numpy as jnp
from jax import lax
from jax.experimental import pallas as pl
from jax.experimental.pallas import tpu as pltpu
from jax.experimental.pallas import tpu_sc as plsc

N = 160000
D = 256
NUM_SEGMENTS = 10000

NUM_CORES = 2
NUM_SUBCORES = 16
DH = D // NUM_CORES
RPT = N // NUM_SUBCORES
RB = 200
NIT = RPT // RB
SCB = 100
NSC = RB // SCB
NCHUNKS = N // RB
ZCH = 200
NZCH = NUM_SEGMENTS // ZCH
KMAX = -(-NZCH // NUM_SUBCORES)


def _seg_sum_body(x_hbm, segs_hbm, out_hbm, acc_sh, xbuf, idxbuf):
    c = lax.axis_index("c")
    s = lax.axis_index("s")
    col0 = c * DH
    zero16 = jnp.zeros((16,), jnp.float32)

    def zero_row(r, carry):
        def zero_lane(j, carry2):
            xbuf[r, pl.ds(j * 16, 16)] = zero16
            return carry2
        return lax.fori_loop(0, DH // 16, zero_lane, carry)

    lax.fori_loop(0, ZCH, zero_row, 0)

    def zero_copy(k, carry):
        ch = s + k * NUM_SUBCORES

        @pl.when(ch < NZCH)
        def _():
            pltpu.sync_copy(xbuf.at[pl.ds(0, ZCH)],
                            acc_sh.at[pl.ds(ch * ZCH, ZCH)])

        return carry

    lax.fori_loop(0, KMAX, zero_copy, 0)
    plsc.subcore_barrier()

    def body(it, carry):
        chunk = s * NIT + it
        pltpu.sync_copy(x_hbm.at[pl.ds(chunk * RB, RB), pl.ds(col0, DH)],
                        xbuf)
        pltpu.sync_copy(segs_hbm.at[chunk], idxbuf)
        for b in range(NSC):
            pltpu.sync_copy(xbuf.at[pl.ds(b * SCB, SCB)],
                            acc_sh.at[idxbuf.at[b]], add=True)
        return carry

    lax.fori_loop(0, NIT, body, 0)
    plsc.subcore_barrier()

    def wb(k, carry):
        ch = s + k * NUM_SUBCORES

        @pl.when(ch < NZCH)
        def _():
            pltpu.sync_copy(acc_sh.at[pl.ds(ch * ZCH, ZCH)],
                            xbuf.at[pl.ds(0, ZCH)])
            pltpu.sync_copy(xbuf.at[pl.ds(0, ZCH)],
                            out_hbm.at[pl.ds(ch * ZCH, ZCH), pl.ds(col0, DH)])

        return carry

    lax.fori_loop(0, KMAX, wb, 0)


@jax.jit
def _seg_sum(xs, segs_r):
    f = pl.kernel(
        _seg_sum_body,
        out_type=jax.ShapeDtypeStruct((NUM_SEGMENTS, D), jnp.float32),
        mesh=plsc.VectorSubcoreMesh(core_axis_name="c", subcore_axis_name="s"),
        scratch_types=[
            pltpu.VMEM_SHARED((NUM_SEGMENTS, DH), jnp.float32),
            pltpu.VMEM((RB, DH), jnp.float32),
            pltpu.VMEM((NSC, SCB), jnp.int32),
        ],
    )
    return f(xs, segs_r)


def kernel(x, segs):
    xs = jnp.squeeze(x, axis=0)
    segs_r = jnp.reshape(segs, (NCHUNKS, NSC, SCB))
    y = _seg_sum(xs, segs_r)
    return jnp.expand_dims(y, axis=0)


# speedup vs baseline: 4.8726x; 1.1938x over previous
import jax
import jax.numpy as jnp
from jax import lax
from jax.experimental import pallas as pl
from jax.experimental.pallas import tpu as pltpu
from jax.experimental.pallas import tpu_sc as plsc

N = 160000
D = 256
NUM_SEGMENTS = 10000

NUM_CORES = 2
NUM_SUBCORES = 16
DH = D // NUM_CORES
RPT = N // NUM_SUBCORES
RB = 80
NIT = RPT // RB
NCHUNKS = N // RB
ZCH = 80
NZCH = NUM_SEGMENTS // ZCH
KMAX = -(-NZCH // NUM_SUBCORES)


def _seg_sum_body(x_hbm, segs_hbm, out_hbm, acc_sh, xb0, xb1, ib0, ib1,
                  sem0, sem1):
    c = lax.axis_index("c")
    s = lax.axis_index("s")
    col0 = c * DH
    zero16 = jnp.zeros((16,), jnp.float32)

    def stage_start(it, xb, ib, sem):
        chunk = s * NIT + it
        pltpu.make_async_copy(
            x_hbm.at[pl.ds(chunk * RB, RB), pl.ds(col0, DH)], xb, sem
        ).start()
        pltpu.make_async_copy(segs_hbm.at[chunk], ib, sem).start()

    def stage_wait(it, xb, ib, sem):
        chunk = s * NIT + it
        pltpu.make_async_copy(
            x_hbm.at[pl.ds(chunk * RB, RB), pl.ds(col0, DH)], xb, sem
        ).wait()
        pltpu.make_async_copy(segs_hbm.at[chunk], ib, sem).wait()

    def scatter(xb, ib):
        pltpu.sync_copy(xb, acc_sh.at[ib.at[0]], add=True)

    stage_start(0, xb0, ib0, sem0)

    def zero_row(r, carry):
        def zero_lane(j, carry2):
            xb1[r, pl.ds(j * 16, 16)] = zero16
            return carry2
        return lax.fori_loop(0, DH // 16, zero_lane, carry)

    lax.fori_loop(0, ZCH, zero_row, 0)

    def zero_copy(k, carry):
        ch = s + k * NUM_SUBCORES

        @pl.when(ch < NZCH)
        def _():
            pltpu.sync_copy(xb1, acc_sh.at[pl.ds(ch * ZCH, ZCH)])

        return carry

    lax.fori_loop(0, KMAX, zero_copy, 0)
    plsc.subcore_barrier()

    def body(g, carry):
        it0 = 2 * g
        stage_wait(it0, xb0, ib0, sem0)
        stage_start(it0 + 1, xb1, ib1, sem1)
        scatter(xb0, ib0)
        stage_wait(it0 + 1, xb1, ib1, sem1)

        @pl.when(it0 + 2 < NIT)
        def _():
            stage_start(it0 + 2, xb0, ib0, sem0)

        scatter(xb1, ib1)
        return carry

    lax.fori_loop(0, NIT // 2, body, 0)
    if NIT % 2 == 1:
        stage_wait(NIT - 1, xb0, ib0, sem0)
        scatter(xb0, ib0)
    plsc.subcore_barrier()

    def wb(k, carry):
        ch = s + k * NUM_SUBCORES

        @pl.when(ch < NZCH)
        def _():
            pltpu.sync_copy(acc_sh.at[pl.ds(ch * ZCH, ZCH)], xb0)
            pltpu.sync_copy(xb0,
                            out_hbm.at[pl.ds(ch * ZCH, ZCH), pl.ds(col0, DH)])

        return carry

    lax.fori_loop(0, KMAX, wb, 0)


@jax.jit
def _seg_sum(xs, segs_r):
    f = pl.kernel(
        _seg_sum_body,
        out_type=jax.ShapeDtypeStruct((NUM_SEGMENTS, D), jnp.float32),
        mesh=plsc.VectorSubcoreMesh(core_axis_name="c", subcore_axis_name="s"),
        scratch_types=[
            pltpu.VMEM_SHARED((NUM_SEGMENTS, DH), jnp.float32),
            pltpu.VMEM((RB, DH), jnp.float32),
            pltpu.VMEM((RB, DH), jnp.float32),
            pltpu.VMEM((1, RB), jnp.int32),
            pltpu.VMEM((1, RB), jnp.int32),
            pltpu.SemaphoreType.DMA,
            pltpu.SemaphoreType.DMA,
        ],
    )
    return f(xs, segs_r)


def kernel(x, segs):
    xs = jnp.squeeze(x, axis=0)
    segs_r = jnp.reshape(segs, (NCHUNKS, 1, RB))
    y = _seg_sum(xs, segs_r)
    return jnp.expand_dims(y, axis=0)
